# baseline (speedup 1.0000x reference)
import functools

import jax
import jax.numpy as jnp
from jax import lax
from jax.experimental import pallas as pl
from jax.experimental.pallas import tpu as pltpu
from jax.experimental.pallas import tpu_sc as plsc

VOCAB = 1_000_000
D = 64
NC = 2
NS = 16
NW = NC * NS
NB = 4096
NT = 200
ROW = 128
NUNITS = (NT // 8) * (NB // ROW) * 8
UPW = NUNITS // NW
GRP = 2
NGRP = UPW // GRP
NPAIR = NGRP // 2


def _embed_body(idx_hbm, w_hbm, out_hbm, raw_slab, idxh0, idxh1,
                selc0, selc1, g_bufs, p_a, p_b,
                gsem_a, gsem_b, wsem_a, wsem_b):
    wid = lax.axis_index("s") * NC + lax.axis_index("c")
    vbase = pl.multiple_of(wid * UPW, 8)
    pltpu.sync_copy(idx_hbm.at[pl.ds(vbase, UPW)], raw_slab)

    lanes = lax.iota(jnp.int32, 16)
    rots = [lax.rem(lanes + k, 16) for k in range(16)]
    gsem = (gsem_a, gsem_b)
    wsem = (wsem_a, wsem_b)
    pbuf = (p_a, p_b)
    idxh = (idxh0, idxh1)
    selc = (selc0, selc1)

    def fire_group(g, s):
        for j in range(GRP):
            for c in range(8):
                sl = pl.ds(c * 16, 16)
                v = raw_slab[g * GRP + j, sl]
                idxh[s][j, sl] = lax.shift_right_logical(v, 1)
                selc[s][j, sl] = lax.shift_left(jnp.bitwise_and(v, 1), 6)
        for j in range(GRP):
            pltpu.async_copy(w_hbm.at[idxh[s].at[j]],
                             g_bufs.at[s * GRP + j], gsem[s])

    def drain_group(s):
        for j in range(GRP):
            pltpu.make_async_copy(w_hbm.at[pl.ds(0, ROW)], g_bufs.at[0],
                                  gsem[s]).wait()

    def transpose(gbuf, pb, selref, j):
        def inner(bb2, _):
            bl_vec = bb2 * 16 + lanes
            sel16 = selref[j, pl.ds(bb2 * 16, 16)]
            for d0 in (0, 16, 32, 48):
                for half in (0, 8):
                    dvs = [jnp.bitwise_or(rots[half + k], d0)
                           for k in range(8)]
                    vals = [plsc.load_gather(gbuf, [bl_vec, dv + sel16])
                            for dv in dvs]
                    for dv, v in zip(dvs, vals):
                        plsc.store_scatter(pb, [dv, bl_vec], v)
            return 0
        lax.fori_loop(0, 8, inner, 0)

    def fire_w(u, p):
        v = vbase + u
        t = (v // 256) * 8 + lax.rem(v, 8)
        bb = lax.rem(v // 8, 32)
        for fr in range(8):
            pltpu.async_copy(pbuf[p].at[pl.ds(fr * 8, 8)],
                             out_hbm.at[t, fr, bb], wsem[p])

    def drain_w(p):
        for fr in range(8):
            pltpu.make_async_copy(pbuf[p].at[pl.ds(fr * 8, 8)],
                                  out_hbm.at[0, fr, 0], wsem[p]).wait()

    def process_unit(u, j, s, k2, guard_first):
        p = j % 2
        if guard_first:
            @pl.when(k2 > 0)
            def _():
                drain_w(p)
        else:
            drain_w(p)
        transpose(g_bufs.at[s * GRP + j], pbuf[p], selc[s], j)
        fire_w(u, p)

    fire_group(0, 0)

    def pair(k2, _):
        g0 = 2 * k2
        drain_group(0)
        fire_group(g0 + 1, 1)
        for j in range(GRP):
            process_unit(g0 * GRP + j, j, 0, k2, True)
        drain_group(1)

        @pl.when(k2 < NPAIR - 1)
        def _():
            fire_group(g0 + 2, 0)

        for j in range(GRP):
            process_unit((g0 + 1) * GRP + j, j, 1, k2, False)
        return 0

    lax.fori_loop(0, NPAIR, pair, 0)
    drain_w(0)
    drain_w(1)


NBLK = VOCAB // ROW
NKPW = NBLK // NW + 1
WSROWS = (NBLK + 1) * D


def _conv_body(wt_hbm, wtail_hbm, ws_hbm, g0, g1, gt, p0, p1,
               gsem0, gsem1, wsem0, wsem1):
    wid = lax.axis_index("s") * NC + lax.axis_index("c")

    lanes = lax.iota(jnp.int32, 16)
    rots = [lax.rem(lanes + k, 16) for k in range(16)]

    def fire_r(bk, gbuf, gsem):
        pltpu.async_copy(wt_hbm.at[:, pl.ds(bk * ROW, ROW)], gbuf, gsem)

    def drain_r(gbuf, gsem):
        pltpu.make_async_copy(wt_hbm.at[:, pl.ds(0, ROW)], gbuf, gsem).wait()

    def transpose(gbuf, pb):
        def inner(bb2, _):
            bl_vec = bb2 * 16 + lanes
            bl_half = lax.shift_right_logical(bl_vec, 1)
            colb = lax.shift_left(jnp.bitwise_and(bl_vec, 1), 6)
            for d0 in (0, 16, 32, 48):
                for half in (0, 8):
                    dvs = [jnp.bitwise_or(rots[half + k], d0)
                           for k in range(8)]
                    vals = [plsc.load_gather(gbuf, [dv, bl_vec])
                            for dv in dvs]
                    for dv, v in zip(dvs, vals):
                        plsc.store_scatter(pb, [bl_half, colb + dv], v)
            return 0
        lax.fori_loop(0, 8, inner, 0)

    def fire_w(bk, pb, wsem):
        pltpu.async_copy(pb, ws_hbm.at[pl.ds(bk * D, D)], wsem)

    def drain_w(pb, wsem):
        pltpu.make_async_copy(pb, ws_hbm.at[pl.ds(0, D)], wsem).wait()

    def step(k, k2, gbuf, gsem, pb, wsem):
        bk = wid + k * NW

        @pl.when(bk < NBLK)
        def _():
            drain_r(gbuf, gsem)

        @pl.when(bk <= NBLK)
        def _():
            @pl.when(k2 > 0)
            def _():
                drain_w(pb, wsem)

        @pl.when(bk < NBLK)
        def _():
            transpose(gbuf, pb)

        @pl.when(bk == NBLK)
        def _():
            pltpu.sync_copy(wtail_hbm, gt)
            transpose(gt, pb)

        @pl.when(wid + (k + 2) * NW < NBLK)
        def _():
            fire_r(wid + (k + 2) * NW, gbuf, gsem)

        @pl.when(bk <= NBLK)
        def _():
            fire_w(bk, pb, wsem)

    fire_r(wid, g0, gsem0)
    fire_r(wid + NW, g1, gsem1)

    def pairs(k2, _):
        step(2 * k2, k2, g0, gsem0, p0, wsem0)
        step(2 * k2 + 1, k2, g1, gsem1, p1, wsem1)
        return 0

    lax.fori_loop(0, (NKPW + 1) // 2, pairs, 0)
    drain_w(p0, wsem0)
    drain_w(p1, wsem1)


_conv = functools.partial(
    pl.kernel,
    mesh=plsc.VectorSubcoreMesh(core_axis_name="c", subcore_axis_name="s"),
    out_type=jax.ShapeDtypeStruct((WSROWS, ROW), jnp.float32),
    scratch_types=[
        pltpu.VMEM((D, ROW), jnp.float32),
        pltpu.VMEM((D, ROW), jnp.float32),
        pltpu.VMEM((D, ROW), jnp.float32),
        pltpu.VMEM((D, ROW), jnp.float32),
        pltpu.VMEM((D, ROW), jnp.float32),
        pltpu.SemaphoreType.DMA,
        pltpu.SemaphoreType.DMA,
        pltpu.SemaphoreType.DMA,
        pltpu.SemaphoreType.DMA,
    ],
    compiler_params=pltpu.CompilerParams(use_tc_tiling_on_sc=True,
                                         needs_layout_passes=False),
)(_conv_body)


_embed = functools.partial(
    pl.kernel,
    mesh=plsc.VectorSubcoreMesh(core_axis_name="c", subcore_axis_name="s"),
    out_type=jax.ShapeDtypeStruct((NT, 8, NB // ROW, 8, ROW), jnp.float32),
    scratch_types=[
        pltpu.VMEM((UPW, ROW), jnp.int32),
        pltpu.VMEM((GRP, ROW), jnp.int32),
        pltpu.VMEM((GRP, ROW), jnp.int32),
        pltpu.VMEM((GRP, ROW), jnp.int32),
        pltpu.VMEM((GRP, ROW), jnp.int32),
        pltpu.VMEM((2 * GRP, ROW, ROW), jnp.float32),
        pltpu.VMEM((D, ROW), jnp.float32),
        pltpu.VMEM((D, ROW), jnp.float32),
        pltpu.SemaphoreType.DMA,
        pltpu.SemaphoreType.DMA,
        pltpu.SemaphoreType.DMA,
        pltpu.SemaphoreType.DMA,
    ],
    compiler_params=pltpu.CompilerParams(use_tc_tiling_on_sc=True,
                                         needs_layout_passes=False),
)(_embed_body)


def kernel(idx, weight):
    idx5 = (idx.T.astype(jnp.int32)
            .reshape(NT // 8, 8, NB // ROW, ROW)
            .transpose(0, 2, 1, 3)
            .reshape(NUNITS, ROW))
    wtail = jnp.pad(weight[VOCAB - D:].T, ((0, 0), (0, ROW - D)))
    ws = _conv(weight.T, wtail)
    out5 = _embed(idx5, ws)
    return (out5.transpose(2, 4, 0, 1, 3)
            .reshape(NB, NT, D))


# speedup vs baseline: 1.0209x; 1.0209x over previous
import functools

import jax
import jax.numpy as jnp
from jax import lax
from jax.experimental import pallas as pl
from jax.experimental.pallas import tpu as pltpu
from jax.experimental.pallas import tpu_sc as plsc

VOCAB = 1_000_000
D = 64
NC = 2
NS = 16
NW = NC * NS
NB = 4096
NT = 200
ROW = 128
NUNITS = (NT // 8) * (NB // ROW) * 8
UPW = NUNITS // NW
GRP = 2
NGRP = UPW // GRP
NPAIR = NGRP // 2


def _embed_body(idx_hbm, w_hbm, out_hbm, raw_slab, g_bufs, p_a, p_b,
                gsem_a, gsem_b, wsem_a, wsem_b):
    wid = lax.axis_index("s") * NC + lax.axis_index("c")
    vbase = pl.multiple_of(wid * UPW, 8)
    pltpu.sync_copy(idx_hbm.at[pl.ds(vbase, UPW)], raw_slab)

    lanes = lax.iota(jnp.int32, 16)
    rots = [lax.rem(lanes + k, 16) for k in range(16)]
    gsem = (gsem_a, gsem_b)
    wsem = (wsem_a, wsem_b)
    pbuf = (p_a, p_b)

    def fire_group(g, s):
        for j in range(GRP):
            pltpu.async_copy(w_hbm.at[raw_slab.at[g * GRP + j]],
                             g_bufs.at[s * GRP + j], gsem[s])

    def drain_group(s):
        for j in range(GRP):
            pltpu.make_async_copy(w_hbm.at[pl.ds(0, ROW)], g_bufs.at[0],
                                  gsem[s]).wait()

    def transpose(gbuf, pb):
        def inner(bb2, _):
            bl_vec = bb2 * 16 + lanes
            for d0 in (0, 16, 32, 48):
                for half in (0, 8):
                    dvs = [jnp.bitwise_or(rots[half + k], d0)
                           for k in range(8)]
                    vals = [plsc.load_gather(gbuf, [bl_vec, dv])
                            for dv in dvs]
                    for dv, v in zip(dvs, vals):
                        plsc.store_scatter(pb, [dv, bl_vec], v)
            return 0
        lax.fori_loop(0, 8, inner, 0)

    def fire_w(u, p):
        v = vbase + u
        t = (v // 256) * 8 + lax.rem(v, 8)
        bb = lax.rem(v // 8, 32)
        for fr in range(8):
            pltpu.async_copy(pbuf[p].at[pl.ds(fr * 8, 8)],
                             out_hbm.at[t, fr, bb], wsem[p])

    def drain_w(p):
        for fr in range(8):
            pltpu.make_async_copy(pbuf[p].at[pl.ds(fr * 8, 8)],
                                  out_hbm.at[0, fr, 0], wsem[p]).wait()

    def process_unit(u, j, s, k2, guard_first):
        p = j % 2
        if guard_first:
            @pl.when(k2 > 0)
            def _():
                drain_w(p)
        else:
            drain_w(p)
        transpose(g_bufs.at[s * GRP + j], pbuf[p])
        fire_w(u, p)

    fire_group(0, 0)

    def pair(k2, _):
        g0 = 2 * k2
        drain_group(0)
        fire_group(g0 + 1, 1)
        for j in range(GRP):
            process_unit(g0 * GRP + j, j, 0, k2, True)
        drain_group(1)

        @pl.when(k2 < NPAIR - 1)
        def _():
            fire_group(g0 + 2, 0)

        for j in range(GRP):
            process_unit((g0 + 1) * GRP + j, j, 1, k2, False)
        return 0

    lax.fori_loop(0, NPAIR, pair, 0)
    drain_w(0)
    drain_w(1)


NBLK = VOCAB // ROW
NKPW = NBLK // NW + 1
VOCABP = (NBLK + 1) * ROW


def _conv_body(wt_hbm, wtail_hbm, ws_hbm, g0, g1, gt, p0, p1,
               gsem0, gsem1, wsem0, wsem1):
    wid = lax.axis_index("s") * NC + lax.axis_index("c")

    lanes = lax.iota(jnp.int32, 16)
    rots = [lax.rem(lanes + k, 16) for k in range(16)]

    def fire_r(bk, gbuf, gsem):
        pltpu.async_copy(wt_hbm.at[:, pl.ds(bk * ROW, ROW)], gbuf, gsem)

    def drain_r(gbuf, gsem):
        pltpu.make_async_copy(wt_hbm.at[:, pl.ds(0, ROW)], gbuf, gsem).wait()

    def transpose(gbuf, pb):
        def inner(bb2, _):
            bl_vec = bb2 * 16 + lanes
            for d0 in (0, 16, 32, 48):
                for half in (0, 8):
                    dvs = [jnp.bitwise_or(rots[half + k], d0)
                           for k in range(8)]
                    vals = [plsc.load_gather(gbuf, [dv, bl_vec])
                            for dv in dvs]
                    for dv, v in zip(dvs, vals):
                        plsc.store_scatter(pb, [bl_vec, dv], v)
            return 0
        lax.fori_loop(0, 8, inner, 0)

    def fire_w(bk, pb, wsem):
        pltpu.async_copy(pb, ws_hbm.at[pl.ds(bk * ROW, ROW)], wsem)

    def drain_w(pb, wsem):
        pltpu.make_async_copy(pb, ws_hbm.at[pl.ds(0, ROW)], wsem).wait()

    def step(k, k2, gbuf, gsem, pb, wsem):
        bk = wid + k * NW

        @pl.when(bk < NBLK)
        def _():
            drain_r(gbuf, gsem)

        @pl.when(bk <= NBLK)
        def _():
            @pl.when(k2 > 0)
            def _():
                drain_w(pb, wsem)

        @pl.when(bk < NBLK)
        def _():
            transpose(gbuf, pb)

        @pl.when(bk == NBLK)
        def _():
            pltpu.sync_copy(wtail_hbm, gt)
            transpose(gt, pb)

        @pl.when(wid + (k + 2) * NW < NBLK)
        def _():
            fire_r(wid + (k + 2) * NW, gbuf, gsem)

        @pl.when(bk <= NBLK)
        def _():
            fire_w(bk, pb, wsem)

    fire_r(wid, g0, gsem0)
    fire_r(wid + NW, g1, gsem1)

    def pairs(k2, _):
        step(2 * k2, k2, g0, gsem0, p0, wsem0)
        step(2 * k2 + 1, k2, g1, gsem1, p1, wsem1)
        return 0

    lax.fori_loop(0, (NKPW + 1) // 2, pairs, 0)
    drain_w(p0, wsem0)
    drain_w(p1, wsem1)


_conv = functools.partial(
    pl.kernel,
    mesh=plsc.VectorSubcoreMesh(core_axis_name="c", subcore_axis_name="s"),
    out_type=jax.ShapeDtypeStruct((VOCABP, ROW), jnp.float32),
    scratch_types=[
        pltpu.VMEM((D, ROW), jnp.float32),
        pltpu.VMEM((D, ROW), jnp.float32),
        pltpu.VMEM((D, ROW), jnp.float32),
        pltpu.VMEM((ROW, ROW), jnp.float32),
        pltpu.VMEM((ROW, ROW), jnp.float32),
        pltpu.SemaphoreType.DMA,
        pltpu.SemaphoreType.DMA,
        pltpu.SemaphoreType.DMA,
        pltpu.SemaphoreType.DMA,
    ],
    compiler_params=pltpu.CompilerParams(use_tc_tiling_on_sc=True,
                                         needs_layout_passes=False),
)(_conv_body)


_embed = functools.partial(
    pl.kernel,
    mesh=plsc.VectorSubcoreMesh(core_axis_name="c", subcore_axis_name="s"),
    out_type=jax.ShapeDtypeStruct((NT, 8, NB // ROW, 8, ROW), jnp.float32),
    scratch_types=[
        pltpu.VMEM((UPW, ROW), jnp.int32),
        pltpu.VMEM((2 * GRP, ROW, ROW), jnp.float32),
        pltpu.VMEM((D, ROW), jnp.float32),
        pltpu.VMEM((D, ROW), jnp.float32),
        pltpu.SemaphoreType.DMA,
        pltpu.SemaphoreType.DMA,
        pltpu.SemaphoreType.DMA,
        pltpu.SemaphoreType.DMA,
    ],
    compiler_params=pltpu.CompilerParams(use_tc_tiling_on_sc=True,
                                         needs_layout_passes=False),
)(_embed_body)


def kernel(idx, weight):
    idx5 = (idx.T.astype(jnp.int32)
            .reshape(NT // 8, 8, NB // ROW, ROW)
            .transpose(0, 2, 1, 3)
            .reshape(NUNITS, ROW))
    wtail = jnp.pad(weight[VOCAB - D:].T, ((0, 0), (0, ROW - D)))
    ws = _conv(weight.T, wtail)
    out5 = _embed(idx5, ws)
    return (out5.transpose(2, 4, 0, 1, 3)
            .reshape(NB, NT, D))
